# baseline (speedup 1.0000x reference)
import jax
import jax.numpy as jnp
from jax import lax
from jax.experimental import pallas as pl
from jax.experimental.pallas import tpu as pltpu

_BB = 8
_R = _BB * 24 * 3


def _dot(a, b):
    return lax.dot(a, b, preferred_element_type=jnp.float32)


def _body(months_ref, ch_ref, pos_ref, mtab_ref, tok_ref, out_ref):
    tok = tok_ref[0]
    n = tok.shape[1] // 4
    m = months_ref[0]
    oh = (m[..., None] == lax.broadcasted_iota(jnp.int32, (_BB, 24, 12), 2))
    oh192 = oh.astype(jnp.float32).reshape(_BB * 24, 12)
    mo192 = _dot(oh192, mtab_ref[...])
    r_i = lax.broadcasted_iota(jnp.int32, (_R, _BB * 24), 0)
    g_i = lax.broadcasted_iota(jnp.int32, (_R, _BB * 24), 1)
    p_g = (r_i // 3 == g_i).astype(jnp.float32)
    mo = _dot(p_g, mo192)
    r3 = lax.broadcasted_iota(jnp.int32, (_R, 3), 0)
    s3 = lax.broadcasted_iota(jnp.int32, (_R, 3), 1)
    p_s = (r3 % 3 == s3).astype(jnp.float32)
    ch = _dot(p_s, ch_ref[...])
    r24 = lax.broadcasted_iota(jnp.int32, (_R, 24), 0)
    t24 = lax.broadcasted_iota(jnp.int32, (_R, 24), 1)
    p_t = (r24 // 3 % 24 == t24).astype(jnp.float32)
    pos = _dot(p_t, pos_ref[...])
    out_ref[0, :, 0:n] = tok[:, 0:n] + ch
    out_ref[0, :, n:2 * n] = tok[:, n:2 * n] + pos
    out_ref[0, :, 2 * n:3 * n] = tok[:, 2 * n:3 * n] + mo
    out_ref[0, :, 3 * n:] = tok[:, 3 * n:]


@jax.jit
def kernel(modality_tokens, timestamps, channel_embed, pos_embed, month_tab):
    b, t, bs, d = modality_tokens.shape
    n = d // 4
    g = b // _BB
    months = timestamps[:, :, 1].astype(jnp.int32).reshape(g, _BB, t)
    tok3 = modality_tokens.reshape(g, _R, d)
    out = pl.pallas_call(
        _body,
        grid=(g,),
        in_specs=[
            pl.BlockSpec((1, _BB, t), lambda i: (i, 0, 0)),
            pl.BlockSpec((bs, n), lambda i: (0, 0)),
            pl.BlockSpec((t, n), lambda i: (0, 0)),
            pl.BlockSpec((12, n), lambda i: (0, 0)),
            pl.BlockSpec((1, _R, d), lambda i: (i, 0, 0)),
        ],
        out_specs=pl.BlockSpec((1, _R, d), lambda i: (i, 0, 0)),
        out_shape=jax.ShapeDtypeStruct((g, _R, d), jnp.float32),
        compiler_params=pltpu.CompilerParams(
            dimension_semantics=("arbitrary",),
            vmem_limit_bytes=100 * 1024 * 1024,
        ),
    )(months, channel_embed, pos_embed[:t], month_tab, tok3)
    return out.reshape(b, t, bs, d)


# speedup vs baseline: 7.2659x; 7.2659x over previous
import jax
import jax.numpy as jnp
from jax.experimental import pallas as pl
from jax.experimental.pallas import tpu as pltpu

_BB = 8


def _body(months_ref, ch_ref, pos_ref, mtab_ref, tok_ref, out_ref):
    tok = tok_ref[...]
    bb, _, t, d = tok.shape
    n = d // 4
    m = months_ref[0]
    mo = jnp.zeros((bb, t, n), jnp.float32)
    for k in range(12):
        sel = (m == k).astype(jnp.float32)[..., None]
        mo = mo + sel * mtab_ref[k, :][None, None, :]
    ch = ch_ref[...]
    pos = pos_ref[...]
    out_ref[..., 0:n] = tok[..., 0:n] + ch[None, :, None, :]
    out_ref[..., n:2 * n] = tok[..., n:2 * n] + pos[None, None, :, :]
    out_ref[..., 2 * n:3 * n] = tok[..., 2 * n:3 * n] + mo[:, None, :, :]
    out_ref[..., 3 * n:] = tok[..., 3 * n:]


@jax.jit
def kernel(modality_tokens, timestamps, channel_embed, pos_embed, month_tab):
    b, t, bs, d = modality_tokens.shape
    n = d // 4
    months = timestamps[:, :, 1].astype(jnp.int32).reshape(b // _BB, _BB, t)
    tok_t = jnp.transpose(modality_tokens, (0, 2, 1, 3))
    out = pl.pallas_call(
        _body,
        grid=(b // _BB,),
        in_specs=[
            pl.BlockSpec((1, _BB, t), lambda i: (i, 0, 0)),
            pl.BlockSpec((bs, n), lambda i: (0, 0)),
            pl.BlockSpec((t, n), lambda i: (0, 0)),
            pl.BlockSpec((12, n), lambda i: (0, 0)),
            pl.BlockSpec((_BB, bs, t, d), lambda i: (i, 0, 0, 0)),
        ],
        out_specs=pl.BlockSpec((_BB, bs, t, d), lambda i: (i, 0, 0, 0)),
        out_shape=jax.ShapeDtypeStruct((b, bs, t, d), jnp.float32),
        compiler_params=pltpu.CompilerParams(
            dimension_semantics=("arbitrary",),
            vmem_limit_bytes=100 * 1024 * 1024,
        ),
    )(months, channel_embed, pos_embed[:t], month_tab, tok_t)
    return jnp.transpose(out, (0, 2, 1, 3))


# speedup vs baseline: 8.0315x; 1.1054x over previous
import jax
import jax.numpy as jnp
from jax.experimental import pallas as pl
from jax.experimental.pallas import tpu as pltpu

_BB = 16


def _body(months_ref, ch_ref, pos_ref, mtab_ref, tok_ref, out_ref):
    tok = tok_ref[...]
    bb, _, t, d = tok.shape
    n = d // 4
    m = months_ref[0]
    mo = jnp.zeros((bb, t, n), jnp.float32)
    for k in range(12):
        sel = (m == k).astype(jnp.float32)[..., None]
        mo = mo + sel * mtab_ref[k, :][None, None, :]
    ch = ch_ref[...]
    pos = pos_ref[...]
    out_ref[..., 0:n] = tok[..., 0:n] + ch[None, :, None, :]
    out_ref[..., n:2 * n] = tok[..., n:2 * n] + pos[None, None, :, :]
    out_ref[..., 2 * n:3 * n] = tok[..., 2 * n:3 * n] + mo[:, None, :, :]
    out_ref[..., 3 * n:] = tok[..., 3 * n:]


@jax.jit
def kernel(modality_tokens, timestamps, channel_embed, pos_embed, month_tab):
    b, t, bs, d = modality_tokens.shape
    n = d // 4
    months = timestamps[:, :, 1].astype(jnp.int32).reshape(b // _BB, _BB, t)
    tok_t = jnp.transpose(modality_tokens, (0, 2, 1, 3))
    out = pl.pallas_call(
        _body,
        grid=(b // _BB,),
        in_specs=[
            pl.BlockSpec((1, _BB, t), lambda i: (i, 0, 0)),
            pl.BlockSpec((bs, n), lambda i: (0, 0)),
            pl.BlockSpec((t, n), lambda i: (0, 0)),
            pl.BlockSpec((12, n), lambda i: (0, 0)),
            pl.BlockSpec((_BB, bs, t, d), lambda i: (i, 0, 0, 0)),
        ],
        out_specs=pl.BlockSpec((_BB, bs, t, d), lambda i: (i, 0, 0, 0)),
        out_shape=jax.ShapeDtypeStruct((b, bs, t, d), jnp.float32),
        compiler_params=pltpu.CompilerParams(
            dimension_semantics=("arbitrary",),
            vmem_limit_bytes=100 * 1024 * 1024,
        ),
    )(months, channel_embed, pos_embed[:t], month_tab, tok_t)
    return jnp.transpose(out, (0, 2, 1, 3))


# speedup vs baseline: 8.0793x; 1.0060x over previous
import jax
import jax.numpy as jnp
from jax.experimental import pallas as pl
from jax.experimental.pallas import tpu as pltpu

_BB = 32


def _body(months_ref, ch_ref, pos_ref, mtab_ref, tok_ref, out_ref):
    tok = tok_ref[...]
    bb, _, t, d = tok.shape
    n = d // 4
    m = months_ref[0]
    mo = jnp.zeros((bb, t, n), jnp.float32)
    for k in range(12):
        sel = (m == k).astype(jnp.float32)[..., None]
        mo = mo + sel * mtab_ref[k, :][None, None, :]
    ch = ch_ref[...]
    pos = pos_ref[...]
    out_ref[..., 0:n] = tok[..., 0:n] + ch[None, :, None, :]
    out_ref[..., n:2 * n] = tok[..., n:2 * n] + pos[None, None, :, :]
    out_ref[..., 2 * n:3 * n] = tok[..., 2 * n:3 * n] + mo[:, None, :, :]
    out_ref[..., 3 * n:] = tok[..., 3 * n:]


@jax.jit
def kernel(modality_tokens, timestamps, channel_embed, pos_embed, month_tab):
    b, t, bs, d = modality_tokens.shape
    n = d // 4
    months = timestamps[:, :, 1].astype(jnp.int32).reshape(b // _BB, _BB, t)
    tok_t = jnp.transpose(modality_tokens, (0, 2, 1, 3))
    out = pl.pallas_call(
        _body,
        grid=(b // _BB,),
        in_specs=[
            pl.BlockSpec((1, _BB, t), lambda i: (i, 0, 0)),
            pl.BlockSpec((bs, n), lambda i: (0, 0)),
            pl.BlockSpec((t, n), lambda i: (0, 0)),
            pl.BlockSpec((12, n), lambda i: (0, 0)),
            pl.BlockSpec((_BB, bs, t, d), lambda i: (i, 0, 0, 0)),
        ],
        out_specs=pl.BlockSpec((_BB, bs, t, d), lambda i: (i, 0, 0, 0)),
        out_shape=jax.ShapeDtypeStruct((b, bs, t, d), jnp.float32),
        compiler_params=pltpu.CompilerParams(
            dimension_semantics=("arbitrary",),
            vmem_limit_bytes=100 * 1024 * 1024,
        ),
    )(months, channel_embed, pos_embed[:t], month_tab, tok_t)
    return jnp.transpose(out, (0, 2, 1, 3))
